# Initial kernel scaffold; baseline (speedup 1.0000x reference)
import jax
import jax.numpy as jnp
from jax.experimental import pallas as pl


def kernel(xyz1, xyz2, points1, points2, W1, b1, g1, be1, W2, b2, g2, be2):
    raise NotImplementedError("write your pallas kernel here")


# speedup vs baseline: 17.5582x; 17.5582x over previous
import functools

import jax
import jax.numpy as jnp
from jax import lax
from jax.experimental import pallas as pl
from jax.experimental.pallas import tpu as pltpu
from jax.experimental.pallas import tpu_sc as plsc


def _topk_body(x1_ref, x2t_ref, idx_ref, w_ref, *, S):
    b = pl.program_id(0)
    x1 = x1_ref[0]
    x2 = x2t_ref[0]
    NT = x1.shape[0]
    d = None
    for c in range(3):
        diff = x1[:, c:c + 1] - x2[c:c + 1, :]
        sq = diff * diff
        d = sq if d is None else d + sq
    iota = lax.broadcasted_iota(jnp.int32, (NT, S), 1)
    idxs, ws = [], []
    for k in range(3):
        m = jnp.min(d, axis=1, keepdims=True)
        ik = jnp.min(jnp.where(d == m, iota, S), axis=1, keepdims=True)
        idxs.append(ik)
        ws.append(1.0 / (m + 1e-8))
        if k < 2:
            d = jnp.where(iota == ik, jnp.float32(jnp.inf), d)
    wcat = jnp.concatenate(ws, axis=1)
    wcat = wcat / jnp.sum(wcat, axis=1, keepdims=True)
    icat = jnp.concatenate(idxs, axis=1) + b * S
    idx_ref[0] = icat
    w_ref[0] = wcat


def _topk(xyz1, xyz2t, NT):
    B, N, _ = xyz1.shape
    S = xyz2t.shape[2]
    grid = (B, N // NT)
    return pl.pallas_call(
        functools.partial(_topk_body, S=S),
        grid=grid,
        in_specs=[
            pl.BlockSpec((1, NT, 3), lambda b, i: (b, i, 0)),
            pl.BlockSpec((1, 3, S), lambda b, i: (b, 0, 0)),
        ],
        out_specs=[
            pl.BlockSpec((1, NT, 3), lambda b, i: (b, i, 0)),
            pl.BlockSpec((1, NT, 3), lambda b, i: (b, i, 0)),
        ],
        out_shape=[
            jax.ShapeDtypeStruct((B, N, 3), jnp.int32),
            jax.ShapeDtypeStruct((B, N, 3), jnp.float32),
        ],
    )(xyz1, xyz2t)


def _sc_gather(table, idx, CH=128):
    M, = idx.shape
    R, C = table.shape
    info = plsc.get_sparse_core_info()
    NW = info.num_cores * info.num_subcores
    n_ch = M // (NW * CH)
    idx2 = idx.reshape(M // CH, CH)
    mesh = plsc.VectorSubcoreMesh(core_axis_name="c", subcore_axis_name="s")

    @functools.partial(
        pl.kernel,
        mesh=mesh,
        out_type=jax.ShapeDtypeStruct((M // CH, CH, C), jnp.float32),
        scratch_types=[
            pltpu.VMEM((CH,), jnp.int32),
            pltpu.VMEM((CH, C), jnp.float32),
            pltpu.SemaphoreType.DMA,
        ],
    )
    def gather_k(idx_hbm, table_hbm, out_hbm, idx_v, rows_v, sem):
        wid = lax.axis_index("c") * info.num_subcores + lax.axis_index("s")

        def body(j, carry):
            blk = wid * n_ch + j
            pltpu.sync_copy(idx_hbm.at[blk], idx_v)
            pltpu.async_copy(table_hbm.at[idx_v], rows_v, sem).wait()
            pltpu.sync_copy(rows_v, out_hbm.at[blk])
            return carry

        lax.fori_loop(0, n_ch, body, 0)

    return gather_k(idx2, table).reshape(M, C)


def _mlp1_body(g_ref, w_ref, p1_ref, W1t_ref, b1_ref, y_ref, s_ref):
    first = (pl.program_id(0) == 0) & (pl.program_id(1) == 0)
    g = g_ref[...]
    w = w_ref[0]
    interp = (g[0, 0] * w[:, 0:1] + g[1, 0] * w[:, 1:2] + g[2, 0] * w[:, 2:3])
    x = jnp.concatenate([p1_ref[0], interp], axis=1)
    y = jnp.dot(x, W1t_ref[...], preferred_element_type=jnp.float32)
    y = y + b1_ref[...]
    y_ref[0] = y
    acc = jnp.concatenate([jnp.sum(y, axis=0, keepdims=True),
                           jnp.sum(y * y, axis=0, keepdims=True)], axis=0)

    @pl.when(first)
    def _():
        s_ref[...] = jnp.zeros_like(s_ref)

    s_ref[...] += acc


def _mlp1(g4, w3, p1t, W1t, b1row, NT):
    _, B, N, C2 = g4.shape
    C1 = p1t.shape[2]
    Co = W1t.shape[1]
    grid = (B, N // NT)
    return pl.pallas_call(
        _mlp1_body,
        grid=grid,
        in_specs=[
            pl.BlockSpec((3, 1, NT, C2), lambda b, i: (0, b, i, 0)),
            pl.BlockSpec((1, NT, 3), lambda b, i: (b, i, 0)),
            pl.BlockSpec((1, NT, C1), lambda b, i: (b, i, 0)),
            pl.BlockSpec((C1 + C2, Co), lambda b, i: (0, 0)),
            pl.BlockSpec((1, Co), lambda b, i: (0, 0)),
        ],
        out_specs=[
            pl.BlockSpec((1, NT, Co), lambda b, i: (b, i, 0)),
            pl.BlockSpec((2, Co), lambda b, i: (0, 0)),
        ],
        out_shape=[
            jax.ShapeDtypeStruct((B, N, Co), jnp.float32),
            jax.ShapeDtypeStruct((2, Co), jnp.float32),
        ],
    )(g4, w3, p1t, W1t, b1row)


def _mlp2_body(y1_ref, sc1_ref, W2t_ref, b2_ref, y2_ref, s_ref):
    first = (pl.program_id(0) == 0) & (pl.program_id(1) == 0)
    sc = sc1_ref[...]
    z = jnp.maximum(y1_ref[0] * sc[0:1, :] + sc[1:2, :], 0.0)
    y2 = jnp.dot(z, W2t_ref[...], preferred_element_type=jnp.float32)
    y2 = y2 + b2_ref[...]
    y2_ref[0] = y2
    acc = jnp.concatenate([jnp.sum(y2, axis=0, keepdims=True),
                           jnp.sum(y2 * y2, axis=0, keepdims=True)], axis=0)

    @pl.when(first)
    def _():
        s_ref[...] = jnp.zeros_like(s_ref)

    s_ref[...] += acc


def _mlp2(y1, sc1, W2t, b2row, NT):
    B, N, Ci = y1.shape
    Co = W2t.shape[1]
    grid = (B, N // NT)
    return pl.pallas_call(
        _mlp2_body,
        grid=grid,
        in_specs=[
            pl.BlockSpec((1, NT, Ci), lambda b, i: (b, i, 0)),
            pl.BlockSpec((2, Ci), lambda b, i: (0, 0)),
            pl.BlockSpec((Ci, Co), lambda b, i: (0, 0)),
            pl.BlockSpec((1, Co), lambda b, i: (0, 0)),
        ],
        out_specs=[
            pl.BlockSpec((1, NT, Co), lambda b, i: (b, i, 0)),
            pl.BlockSpec((2, Co), lambda b, i: (0, 0)),
        ],
        out_shape=[
            jax.ShapeDtypeStruct((B, N, Co), jnp.float32),
            jax.ShapeDtypeStruct((2, Co), jnp.float32),
        ],
    )(y1, sc1, W2t, b2row)


def _bn2_body(y2_ref, sc2_ref, o_ref):
    sc = sc2_ref[...]
    o_ref[0] = jnp.maximum(y2_ref[0] * sc[0:1, :] + sc[1:2, :], 0.0)


def _bn2(y2, sc2, NT):
    B, N, Co = y2.shape
    grid = (B, N // NT)
    return pl.pallas_call(
        _bn2_body,
        grid=grid,
        in_specs=[
            pl.BlockSpec((1, NT, Co), lambda b, i: (b, i, 0)),
            pl.BlockSpec((2, Co), lambda b, i: (0, 0)),
        ],
        out_specs=pl.BlockSpec((1, NT, Co), lambda b, i: (b, i, 0)),
        out_shape=jax.ShapeDtypeStruct((B, N, Co), jnp.float32),
    )(y2, sc2)


def _bn_scale_shift(sums, count, g, be):
    mean = sums[0] / count
    var = sums[1] / count - mean * mean
    scale = g * lax.rsqrt(var + 1e-5)
    shift = be - mean * scale
    return jnp.stack([scale, shift])


def kernel(xyz1, xyz2, points1, points2, W1, b1, g1, be1, W2, b2, g2, be2):
    B, N, _ = xyz1.shape
    S = xyz2.shape[1]
    C1 = points1.shape[1]
    C2 = points2.shape[1]
    NT = 512

    xyz2t = jnp.transpose(xyz2, (0, 2, 1))
    idx3, w3 = _topk(xyz1, xyz2t, NT)

    pts2 = jnp.transpose(points2, (0, 2, 1)).reshape(B * S, C2)
    idx_flat = jnp.transpose(idx3, (2, 0, 1)).reshape(-1)
    gathered = _sc_gather(pts2, idx_flat)
    g4 = gathered.reshape(3, B, N, C2)

    p1t = jnp.transpose(points1, (0, 2, 1))
    y1, s1 = _mlp1(g4, w3, p1t, jnp.transpose(W1), b1[None, :], NT)
    sc1 = _bn_scale_shift(s1, B * N, g1, be1)
    y2, s2 = _mlp2(y1, sc1, jnp.transpose(W2), b2[None, :], NT)
    sc2 = _bn_scale_shift(s2, B * N, g2, be2)
    outt = _bn2(y2, sc2, NT)
    return jnp.transpose(outt, (0, 2, 1))


# speedup vs baseline: 19.2544x; 1.0966x over previous
import functools

import jax
import jax.numpy as jnp
from jax import lax
from jax.experimental import pallas as pl
from jax.experimental.pallas import tpu as pltpu
from jax.experimental.pallas import tpu_sc as plsc


def _topk_body(x1_ref, x2t_ref, idx_ref, w_ref, *, S):
    b = pl.program_id(0)
    x1 = x1_ref[0]
    x2 = x2t_ref[0]
    NT = x1.shape[0]
    cross = lax.dot_general(x1, x2, (((1,), (0,)), ((), ())),
                            preferred_element_type=jnp.float32)
    n1 = jnp.sum(x1 * x1, axis=1, keepdims=True)
    n2 = jnp.sum(x2 * x2, axis=0, keepdims=True)
    d = jnp.maximum(n1 - 2.0 * cross + n2, 0.0)
    iota_f = lax.broadcasted_iota(jnp.int32, (NT, S), 1).astype(jnp.float32)
    idxs, ws = [], []
    for k in range(3):
        m = jnp.min(d, axis=1, keepdims=True)
        ikf = jnp.min(jnp.where(d == m, iota_f, jnp.float32(S)),
                      axis=1, keepdims=True)
        idxs.append(ikf.astype(jnp.int32))
        ws.append(1.0 / (m + 1e-8))
        if k < 2:
            d = jnp.where(iota_f == ikf, jnp.float32(jnp.inf), d)
    wcat = jnp.concatenate(ws, axis=1)
    wcat = wcat / jnp.sum(wcat, axis=1, keepdims=True)
    icat = jnp.concatenate(idxs, axis=1) + b * S
    idx_ref[0] = icat
    w_ref[0] = wcat


def _topk(xyz1, xyz2t, NT):
    B, N, _ = xyz1.shape
    S = xyz2t.shape[2]
    grid = (B, N // NT)
    return pl.pallas_call(
        functools.partial(_topk_body, S=S),
        grid=grid,
        in_specs=[
            pl.BlockSpec((1, NT, 3), lambda b, i: (b, i, 0)),
            pl.BlockSpec((1, 3, S), lambda b, i: (b, 0, 0)),
        ],
        out_specs=[
            pl.BlockSpec((1, NT, 3), lambda b, i: (b, i, 0)),
            pl.BlockSpec((1, NT, 3), lambda b, i: (b, i, 0)),
        ],
        out_shape=[
            jax.ShapeDtypeStruct((B, N, 3), jnp.int32),
            jax.ShapeDtypeStruct((B, N, 3), jnp.float32),
        ],
    )(xyz1, xyz2t)


def _sc_gather(table, idx, CH=128):
    M, = idx.shape
    R, C = table.shape
    info = plsc.get_sparse_core_info()
    NW = info.num_cores * info.num_subcores
    n_ch = M // (NW * CH)
    idx2 = idx.reshape(M // CH, CH)
    mesh = plsc.VectorSubcoreMesh(core_axis_name="c", subcore_axis_name="s")

    @functools.partial(
        pl.kernel,
        mesh=mesh,
        out_type=jax.ShapeDtypeStruct((M // CH, CH, C), jnp.float32),
        scratch_types=[
            pltpu.VMEM((CH,), jnp.int32),
            pltpu.VMEM((CH, C), jnp.float32),
            pltpu.SemaphoreType.DMA,
        ],
    )
    def gather_k(idx_hbm, table_hbm, out_hbm, idx_v, rows_v, sem):
        wid = lax.axis_index("c") * info.num_subcores + lax.axis_index("s")

        def body(j, carry):
            blk = wid * n_ch + j
            pltpu.sync_copy(idx_hbm.at[blk], idx_v)
            pltpu.async_copy(table_hbm.at[idx_v], rows_v, sem).wait()
            pltpu.sync_copy(rows_v, out_hbm.at[blk])
            return carry

        lax.fori_loop(0, n_ch, body, 0)

    return gather_k(idx2, table).reshape(M, C)


def _mlp1_body(g_ref, w_ref, p1_ref, W1t_ref, b1_ref, y_ref, s_ref):
    first = (pl.program_id(0) == 0) & (pl.program_id(1) == 0)
    g = g_ref[...]
    w = w_ref[0]
    interp = (g[0, 0] * w[:, 0:1] + g[1, 0] * w[:, 1:2] + g[2, 0] * w[:, 2:3])
    x = jnp.concatenate([p1_ref[0], interp], axis=1)
    y = jnp.dot(x, W1t_ref[...], preferred_element_type=jnp.float32)
    y = y + b1_ref[...]
    y_ref[0] = y
    acc = jnp.concatenate([jnp.sum(y, axis=0, keepdims=True),
                           jnp.sum(y * y, axis=0, keepdims=True)], axis=0)

    @pl.when(first)
    def _():
        s_ref[...] = jnp.zeros_like(s_ref)

    s_ref[...] += acc


def _mlp1(g4, w3, p1t, W1t, b1row, NT):
    _, B, N, C2 = g4.shape
    C1 = p1t.shape[2]
    Co = W1t.shape[1]
    grid = (B, N // NT)
    return pl.pallas_call(
        _mlp1_body,
        grid=grid,
        in_specs=[
            pl.BlockSpec((3, 1, NT, C2), lambda b, i: (0, b, i, 0)),
            pl.BlockSpec((1, NT, 3), lambda b, i: (b, i, 0)),
            pl.BlockSpec((1, NT, C1), lambda b, i: (b, i, 0)),
            pl.BlockSpec((C1 + C2, Co), lambda b, i: (0, 0)),
            pl.BlockSpec((1, Co), lambda b, i: (0, 0)),
        ],
        out_specs=[
            pl.BlockSpec((1, NT, Co), lambda b, i: (b, i, 0)),
            pl.BlockSpec((2, Co), lambda b, i: (0, 0)),
        ],
        out_shape=[
            jax.ShapeDtypeStruct((B, N, Co), jnp.float32),
            jax.ShapeDtypeStruct((2, Co), jnp.float32),
        ],
    )(g4, w3, p1t, W1t, b1row)


def _mlp2_body(y1_ref, sc1_ref, W2t_ref, b2_ref, y2_ref, s_ref):
    first = (pl.program_id(0) == 0) & (pl.program_id(1) == 0)
    sc = sc1_ref[...]
    z = jnp.maximum(y1_ref[0] * sc[0:1, :] + sc[1:2, :], 0.0)
    y2 = jnp.dot(z, W2t_ref[...], preferred_element_type=jnp.float32)
    y2 = y2 + b2_ref[...]
    y2_ref[0] = y2
    acc = jnp.concatenate([jnp.sum(y2, axis=0, keepdims=True),
                           jnp.sum(y2 * y2, axis=0, keepdims=True)], axis=0)

    @pl.when(first)
    def _():
        s_ref[...] = jnp.zeros_like(s_ref)

    s_ref[...] += acc


def _mlp2(y1, sc1, W2t, b2row, NT):
    B, N, Ci = y1.shape
    Co = W2t.shape[1]
    grid = (B, N // NT)
    return pl.pallas_call(
        _mlp2_body,
        grid=grid,
        in_specs=[
            pl.BlockSpec((1, NT, Ci), lambda b, i: (b, i, 0)),
            pl.BlockSpec((2, Ci), lambda b, i: (0, 0)),
            pl.BlockSpec((Ci, Co), lambda b, i: (0, 0)),
            pl.BlockSpec((1, Co), lambda b, i: (0, 0)),
        ],
        out_specs=[
            pl.BlockSpec((1, NT, Co), lambda b, i: (b, i, 0)),
            pl.BlockSpec((2, Co), lambda b, i: (0, 0)),
        ],
        out_shape=[
            jax.ShapeDtypeStruct((B, N, Co), jnp.float32),
            jax.ShapeDtypeStruct((2, Co), jnp.float32),
        ],
    )(y1, sc1, W2t, b2row)


def _bn2_body(y2_ref, sc2_ref, o_ref):
    sc = sc2_ref[...]
    o_ref[0] = jnp.maximum(y2_ref[0] * sc[0:1, :] + sc[1:2, :], 0.0)


def _bn2(y2, sc2, NT):
    B, N, Co = y2.shape
    grid = (B, N // NT)
    return pl.pallas_call(
        _bn2_body,
        grid=grid,
        in_specs=[
            pl.BlockSpec((1, NT, Co), lambda b, i: (b, i, 0)),
            pl.BlockSpec((2, Co), lambda b, i: (0, 0)),
        ],
        out_specs=pl.BlockSpec((1, NT, Co), lambda b, i: (b, i, 0)),
        out_shape=jax.ShapeDtypeStruct((B, N, Co), jnp.float32),
    )(y2, sc2)


def _bn_scale_shift(sums, count, g, be):
    mean = sums[0] / count
    var = sums[1] / count - mean * mean
    scale = g * lax.rsqrt(var + 1e-5)
    shift = be - mean * scale
    return jnp.stack([scale, shift])


def kernel(xyz1, xyz2, points1, points2, W1, b1, g1, be1, W2, b2, g2, be2):
    B, N, _ = xyz1.shape
    S = xyz2.shape[1]
    C1 = points1.shape[1]
    C2 = points2.shape[1]
    NT = 512

    xyz2t = jnp.transpose(xyz2, (0, 2, 1))
    idx3, w3 = _topk(xyz1, xyz2t, NT)

    pts2 = jnp.transpose(points2, (0, 2, 1)).reshape(B * S, C2)
    idx_flat = jnp.transpose(idx3, (2, 0, 1)).reshape(-1)
    gathered = _sc_gather(pts2, idx_flat)
    g4 = gathered.reshape(3, B, N, C2)

    p1t = jnp.transpose(points1, (0, 2, 1))
    y1, s1 = _mlp1(g4, w3, p1t, jnp.transpose(W1), b1[None, :], NT)
    sc1 = _bn_scale_shift(s1, B * N, g1, be1)
    y2, s2 = _mlp2(y1, sc1, jnp.transpose(W2), b2[None, :], NT)
    sc2 = _bn_scale_shift(s2, B * N, g2, be2)
    outt = _bn2(y2, sc2, NT)
    return jnp.transpose(outt, (0, 2, 1))
